# Initial kernel scaffold; baseline (speedup 1.0000x reference)
import jax
import jax.numpy as jnp
from jax.experimental import pallas as pl


def kernel(T, table):
    raise NotImplementedError("write your pallas kernel here")


# speedup vs baseline: 42.2981x; 42.2981x over previous
import functools

import jax
import jax.numpy as jnp
from jax import lax
from jax.experimental import pallas as pl
from jax.experimental.pallas import tpu as pltpu
from jax.experimental.pallas import tpu_sc as plsc

MAX_LEN = 2048
NUM_HEADS = 16
PAD_W = 2 * MAX_LEN
NSHIFT = 8
NBUF = 8

_info = plsc.get_sparse_core_info()
_NC, _NS = _info.num_cores, _info.num_subcores
_NW = _NC * _NS
_ROWS_PER = (NUM_HEADS * MAX_LEN) // _NW
_WPH = MAX_LEN // _ROWS_PER


def _make_sc_kernel():
    mesh = plsc.VectorSubcoreMesh(core_axis_name="c", subcore_axis_name="s")

    @functools.partial(
        pl.kernel,
        mesh=mesh,
        out_type=jax.ShapeDtypeStruct((NUM_HEADS * MAX_LEN * MAX_LEN,), jnp.float32),
        scratch_types=[pltpu.VMEM((PAD_W,), jnp.float32)] * NSHIFT
        + [pltpu.SemaphoreType.DMA] * NBUF,
    )
    def sc_bias(tt8_hbm, out_hbm, *scratch):
        vs, sems = scratch[:NSHIFT], scratch[NSHIFT:]
        wid = lax.axis_index("s") * _NC + lax.axis_index("c")
        h = wid // _WPH
        i0 = (wid % _WPH) * _ROWS_PER

        for s in range(NSHIFT):
            pltpu.sync_copy(tt8_hbm.at[pl.ds((h * NSHIFT + s) * PAD_W, PAD_W)], vs[s])

        kmax = _ROWS_PER // NSHIFT

        for r in range(NSHIFT):
            s_r = (MAX_LEN - 1 - r) % NSHIFT
            base = MAX_LEN - 1 - s_r - r - i0

            def blk(g, carry, r=r, s_r=s_r, base=base):
                for b in range(NBUF):
                    k = g * NBUF + b
                    i = i0 + r + NSHIFT * k
                    off = pl.multiple_of(base - NSHIFT * k, NSHIFT)
                    src = vs[s_r].at[pl.ds(off, MAX_LEN)]
                    dst = out_hbm.at[pl.ds((h * MAX_LEN + i) * MAX_LEN, MAX_LEN)]

                    @pl.when(g > 0)
                    def _wait():
                        pltpu.make_async_copy(src, dst, sems[b]).wait()

                    pltpu.make_async_copy(src, dst, sems[b]).start()
                return carry

            lax.fori_loop(0, kmax // NBUF, blk, 0)

            for b in range(NBUF):
                pltpu.make_async_copy(
                    vs[s_r].at[pl.ds(0, MAX_LEN)],
                    out_hbm.at[pl.ds((h * MAX_LEN + i0) * MAX_LEN, MAX_LEN)],
                    sems[b],
                ).wait()

    return sc_bias


_sc_bias = _make_sc_kernel()


@jax.jit
def kernel(T, table):
    del T
    ttp = jnp.pad(jnp.transpose(table), ((0, 0), (0, NSHIFT + 1)))
    tt8 = jnp.stack(
        [ttp[:, s : s + PAD_W] for s in range(NSHIFT)], axis=1
    )
    out = _sc_bias(tt8.reshape(-1))
    return out.reshape(NUM_HEADS, MAX_LEN, MAX_LEN)
